# baseline (speedup 1.0000x reference)
import functools

import jax
import jax.numpy as jnp
from jax import lax
from jax.experimental import pallas as pl
from jax.experimental.pallas import tpu as pltpu, tpu_sc as plsc

L = 16
NC, NS = 2, 16
NW = NC * NS
CH = 16


@functools.lru_cache(maxsize=None)
def _build(num_boxes: int, batch: int, dim: int):
    bpw = batch // NW
    ndg = dim // 8
    mesh = plsc.VectorSubcoreMesh(core_axis_name="c", subcore_axis_name="s")

    @functools.partial(
        pl.kernel,
        mesh=mesh,
        compiler_params=pltpu.CompilerParams(needs_layout_passes=False),
        out_type=jax.ShapeDtypeStruct((2, ndg, 8, batch), jnp.float32),
        scratch_types=[
            pltpu.VMEM((bpw,), jnp.int32),
            pltpu.VMEM((ndg, 8, CH * 128), jnp.float32),
            pltpu.VMEM((2, ndg, 8, bpw), jnp.float32),
            pltpu.SemaphoreType.DMA,
        ],
    )
    def deltabox(zt_hbm, ldt_hbm, ids_hbm, out_hbm, idx_v, win_v, out_v, sem):
        wid = lax.axis_index("s") * NC + lax.axis_index("c")
        base = pl.multiple_of(wid * bpw, 128)
        pltpu.sync_copy(ids_hbm.at[pl.ds(base, bpw)], idx_v)

        r16 = lax.iota(jnp.int32, L)
        dg_a = r16 >> 3
        dg_b = dg_a + 2
        dr_ab = r16 & 7
        zero16 = jnp.zeros((L,), jnp.int32)
        one16 = zero16 + 1

        def make_pass(tab_hbm, is_ld):
            def chunk_body(c, carry):
                vecb = idx_v[pl.ds(c * CH, CH)]
                for j in range(CH):
                    b = vecb[j]
                    bg = pl.multiple_of((b >> 7) * 128, 128)
                    pltpu.make_async_copy(
                        tab_hbm.at[:, :, pl.ds(bg, 128)],
                        win_v.at[:, :, pl.ds(j * 128, 128)],
                        sem).start()
                pltpu.make_async_copy(tab_hbm.at[:, :, pl.ds(0, CH * 128)],
                                      win_v, sem).wait()
                for j in range(CH):
                    b = vecb[j]
                    k = c * CH + j
                    colv = jnp.full((L,), b & 127, jnp.int32) + (j * 128)
                    kv = jnp.full((L,), k, jnp.int32)
                    ga = plsc.load_gather(win_v, [dg_a, dr_ab, colv])
                    gb = plsc.load_gather(win_v, [dg_b, dr_ab, colv])
                    if not is_ld:
                        plsc.store_scatter(out_v, [zero16, dg_a, dr_ab, kv],
                                           ga)
                        plsc.store_scatter(out_v, [zero16, dg_b, dr_ab, kv],
                                           gb)
                    else:
                        mna = plsc.load_gather(out_v, [zero16, dg_a, dr_ab,
                                                       kv])
                        mnb = plsc.load_gather(out_v, [zero16, dg_b, dr_ab,
                                                       kv])
                        plsc.store_scatter(out_v, [one16, dg_a, dr_ab, kv],
                                           mna + jnp.exp(ga))
                        plsc.store_scatter(out_v, [one16, dg_b, dr_ab, kv],
                                           mnb + jnp.exp(gb))
                return carry
            return chunk_body

        lax.fori_loop(0, bpw // CH, make_pass(zt_hbm, False), 0)
        lax.fori_loop(0, bpw // CH, make_pass(ldt_hbm, True), 0)
        pltpu.sync_copy(out_v, out_hbm.at[:, :, :, pl.ds(base, bpw)])

    return deltabox


def kernel(z, logdelta, ids):
    num_models, num_boxes, dim = z.shape
    batch = ids.shape[0]
    fn = _build(num_boxes, batch, dim)
    zt = jnp.swapaxes(z, 1, 2).reshape(dim // 8, 8, num_boxes)
    ldt = jnp.swapaxes(logdelta, 1, 2).reshape(dim // 8, 8, num_boxes)
    out = fn(zt, ldt, ids.astype(jnp.int32))
    return jnp.transpose(out, (3, 0, 1, 2)).reshape(1, batch, 2, dim)


# speedup vs baseline: 1.0889x; 1.0889x over previous
import functools

import jax
import jax.numpy as jnp
from jax import lax
from jax.experimental import pallas as pl
from jax.experimental.pallas import tpu as pltpu, tpu_sc as plsc

L = 16
NC, NS = 2, 16
NW = NC * NS
CH = 8


@functools.lru_cache(maxsize=None)
def _build(num_boxes: int, batch: int, dim: int):
    bpw = batch // NW
    ndg = dim // 8
    mesh = plsc.VectorSubcoreMesh(core_axis_name="c", subcore_axis_name="s")

    @functools.partial(
        pl.kernel,
        mesh=mesh,
        compiler_params=pltpu.CompilerParams(needs_layout_passes=False),
        out_type=jax.ShapeDtypeStruct((2, ndg, 8, batch), jnp.float32),
        scratch_types=[
            pltpu.VMEM((bpw + L,), jnp.int32),
            pltpu.VMEM((2, ndg, 8, CH * 128), jnp.float32),
            pltpu.VMEM((2, ndg, 8, bpw), jnp.float32),
            pltpu.SemaphoreType.DMA,
            pltpu.SemaphoreType.DMA,
        ],
    )
    def deltabox(zt_hbm, ldt_hbm, ids_hbm, out_hbm, idx_v, win_v, out_v,
                 sem0, sem1):
        wid = lax.axis_index("s") * NC + lax.axis_index("c")
        base = pl.multiple_of(wid * bpw, 128)
        pltpu.sync_copy(ids_hbm.at[pl.ds(base, bpw)], idx_v.at[pl.ds(0, bpw)])

        r16 = lax.iota(jnp.int32, L)
        dg_a = r16 >> 3
        dg_b = dg_a + 2
        dr_ab = r16 & 7
        zero16 = jnp.zeros((L,), jnp.int32)
        one16 = zero16 + 1
        nch = bpw // CH

        def make_pass(tab_hbm, is_ld):
            def issue(c, buf, sem):
                c = jnp.minimum(c, nch - 1)
                vecb = idx_v[pl.ds(c * CH, L)]
                for j in range(CH):
                    b = vecb[j]
                    bg = pl.multiple_of((b >> 7) * 128, 128)
                    pltpu.make_async_copy(
                        tab_hbm.at[:, :, pl.ds(bg, 128)],
                        win_v.at[buf, :, :, pl.ds(j * 128, 128)],
                        sem).start()

            def drain(buf, sem):
                pltpu.make_async_copy(tab_hbm.at[:, :, pl.ds(0, CH * 128)],
                                      win_v.at[buf], sem).wait()

            def extract(c, buf):
                vecb = idx_v[pl.ds(c * CH, L)]
                for j in range(CH):
                    b = vecb[j]
                    k = c * CH + j
                    colv = jnp.full((L,), b & 127, jnp.int32) + (j * 128)
                    kv = jnp.full((L,), k, jnp.int32)
                    ga = plsc.load_gather(win_v.at[buf], [dg_a, dr_ab, colv])
                    gb = plsc.load_gather(win_v.at[buf], [dg_b, dr_ab, colv])
                    if not is_ld:
                        plsc.store_scatter(out_v, [zero16, dg_a, dr_ab, kv],
                                           ga)
                        plsc.store_scatter(out_v, [zero16, dg_b, dr_ab, kv],
                                           gb)
                    else:
                        mna = plsc.load_gather(out_v, [zero16, dg_a, dr_ab,
                                                       kv])
                        mnb = plsc.load_gather(out_v, [zero16, dg_b, dr_ab,
                                                       kv])
                        plsc.store_scatter(out_v, [one16, dg_a, dr_ab, kv],
                                           mna + jnp.exp(ga))
                        plsc.store_scatter(out_v, [one16, dg_b, dr_ab, kv],
                                           mnb + jnp.exp(gb))

            def pair_body(p, carry):
                c0 = p * 2
                issue(c0 + 1, 1, sem1)
                drain(0, sem0)
                extract(c0, 0)
                issue(c0 + 2, 0, sem0)
                drain(1, sem1)
                extract(c0 + 1, 1)
                return carry

            issue(0, 0, sem0)
            lax.fori_loop(0, nch // 2, pair_body, 0)
            drain(0, sem0)

        make_pass(zt_hbm, False)
        make_pass(ldt_hbm, True)
        pltpu.sync_copy(out_v, out_hbm.at[:, :, :, pl.ds(base, bpw)])

    return deltabox


def kernel(z, logdelta, ids):
    num_models, num_boxes, dim = z.shape
    batch = ids.shape[0]
    fn = _build(num_boxes, batch, dim)
    zt = jnp.swapaxes(z, 1, 2).reshape(dim // 8, 8, num_boxes)
    ldt = jnp.swapaxes(logdelta, 1, 2).reshape(dim // 8, 8, num_boxes)
    out = fn(zt, ldt, ids.astype(jnp.int32))
    return jnp.transpose(out, (3, 0, 1, 2)).reshape(1, batch, 2, dim)
